# Initial kernel scaffold; baseline (speedup 1.0000x reference)
import jax
import jax.numpy as jnp
from jax.experimental import pallas as pl


def kernel(hidden_states, weight):
    raise NotImplementedError("write your pallas kernel here")


# speedup vs baseline: 3.1988x; 3.1988x over previous
import jax
import jax.numpy as jnp
from jax.experimental import pallas as pl

_TOPK = 8
_NE = 64
_NG = 8
_TG = 4
_SCALE = 16.0


def _gate_block(x_ref, w_ref, idx_ref, wgt_ref):
    x = x_ref[...]
    w = w_ref[...]
    logits = jax.lax.dot_general(
        x, w, (((1,), (1,)), ((), ())),
        preferred_element_type=jnp.float32,
        precision=jax.lax.Precision.DEFAULT,
    )
    lt = logits.T
    m = jnp.max(lt, axis=0, keepdims=True)
    e = jnp.exp(lt - m)
    s = jnp.sum(e, axis=0, keepdims=True)
    scores = e / s

    bt = scores.shape[-1]
    g = scores.reshape(_NG, _NE // _NG, bt)
    gmax = jnp.max(g, axis=1)
    giota = jax.lax.broadcasted_iota(jnp.int32, gmax.shape, 0).astype(
        jnp.float32)
    gv = gmax
    for _ in range(_TG):
        gm = jnp.max(gv, axis=0, keepdims=True)
        gi = jnp.min(jnp.where(gv == gm, giota, float(_NG)), axis=0,
                     keepdims=True)
        gv = jnp.where(giota == gi, -1.0, gv)
    gsel = gv == -1.0

    mask = jnp.broadcast_to(gsel[:, None, :], (_NG, _NE // _NG, bt))
    mask = mask.reshape(_NE, bt)
    t = jnp.where(mask, scores, 0.0)

    eiota = jax.lax.broadcasted_iota(jnp.int32, t.shape, 0).astype(
        jnp.float32)
    idxs, wgts = [], []
    for _ in range(_TOPK):
        m2 = jnp.max(t, axis=0, keepdims=True)
        ei = jnp.min(jnp.where(t == m2, eiota, float(_NE)), axis=0,
                     keepdims=True)
        idxs.append(ei)
        wgts.append(m2 * _SCALE)
        t = jnp.where(eiota == ei, -1.0, t)

    idx_t = jnp.concatenate(idxs, axis=0)
    wgt_t = jnp.concatenate(wgts, axis=0)
    idx_ref[...] = idx_t.T.astype(jnp.int32)
    wgt_ref[...] = wgt_t.T


def kernel(hidden_states, weight):
    b, s, h = hidden_states.shape
    x = hidden_states.reshape(-1, h)
    n = x.shape[0]
    bt = 256
    idx, wgt = pl.pallas_call(
        _gate_block,
        grid=(n // bt,),
        in_specs=[
            pl.BlockSpec((bt, h), lambda i: (i, 0)),
            pl.BlockSpec((_NE, h), lambda i: (0, 0)),
        ],
        out_specs=[
            pl.BlockSpec((bt, _TOPK), lambda i: (i, 0)),
            pl.BlockSpec((bt, _TOPK), lambda i: (i, 0)),
        ],
        out_shape=[
            jax.ShapeDtypeStruct((n, _TOPK), jnp.int32),
            jax.ShapeDtypeStruct((n, _TOPK), jnp.float32),
        ],
    )(x, weight)
    return idx, wgt
